# baseline (speedup 1.0000x reference)
import functools

import numpy as np
import jax
import jax.numpy as jnp
from jax import lax
from jax.experimental import pallas as pl
from jax.experimental.pallas import tpu as pltpu
from jax.experimental.pallas import tpu_sc as plsc

F32 = jnp.float32
LN = 16
P = 8
NW = 32
NS = 16


def _dot(a, b):
    return lax.dot_general(a, b, (((1,), (0,)), ((), ())),
                           preferred_element_type=F32)


def _pdot(a, b):
    return lax.dot_general(a, b, (((1,), (0,)), ((), ())),
                           preferred_element_type=F32,
                           precision=lax.Precision.HIGHEST)


def _b16(x):
    return x.astype(jnp.bfloat16).astype(F32)


def _np_px():
    m = np.zeros((128, 32), np.float32)
    for j in range(P):
        for k in range(3):
            m[16 * j + k, 4 * j + k] = 1.0
    return m


def _np_pxt():
    m = np.zeros((32, 128), np.float32)
    for j in range(P):
        for k in range(3):
            m[4 * j + k, 16 * j + 3 + k] = 1.0
    return m


def _np_shvx():
    m = np.zeros((128, 128), np.float32)
    for j in range(P):
        for k in range(3):
            m[16 * j + 3 + k, 16 * j + k] = 1.0
    return m


def _np_gxv():
    m = np.zeros((128, 128), np.float32)
    for j in range(P):
        for a in range(3):
            for b in range(3):
                m[16 * j + a, 16 * j + b] = 1.0
                m[16 * j + 3 + a, 16 * j + 3 + b] = 1.0
    return m


def _np_gxall():
    m = np.zeros((128, 128), np.float32)
    for j in range(P):
        for a in range(3):
            for b in range(16):
                m[16 * j + a, 16 * j + b] = 1.0
    return m


def _np_gh():
    m = np.zeros((128, 128), np.float32)
    for j in range(P):
        for a in range(6, 16):
            for b in range(6, 16):
                m[16 * j + a, 16 * j + b] = 0.1
    return m


_PX = _np_px()
_GXALL = _np_gxall()
_PXT = _np_pxt()
_SHVX = _np_shvx()
_GXV = _np_gxv()
_GH = _np_gh()


def _bd(w16):
    return jnp.kron(jnp.eye(P, dtype=F32), w16.astype(F32))


def _tile(b16):
    return jnp.tile(b16.astype(F32), P)[None]


def _edge_weights(sp, gvec):
    e0, e1, e2 = sp["phi_e"]
    x0, x1, x2 = sp["phi_x"]
    W1 = e0["W"]
    z = jnp.zeros((16, 16), F32)
    wa = _bd(z.at[6:16].set(W1[0:10]))
    wb = _bd(z.at[6:16].set(W1[10:20]))
    wdt = _tile(_b16(W1[20]))
    gxall = jnp.asarray(_GXALL)
    c0 = _tile(e0["b"] + gvec @ W1[21:25])
    w2, b2 = _bd(e1["W"]), _tile(e1["b"])
    w3, b3 = _bd(e2["W"]), _tile(e2["b"])
    wx1, bx1 = _bd(x0["W"]), _tile(x0["b"])
    wx2, bx2 = _bd(x1["W"]), _tile(x1["b"])
    ws = _bd(z.at[:, 0:3].set(jnp.broadcast_to(x2["W"], (16, 3))))
    bs = _tile(jnp.zeros((16,), F32).at[0:3].set(x2["b"][0]))
    px = jnp.asarray(_PX)
    return (wa, wb, wdt, gxall, c0, w2, b2, w3, b3, wx1, bx1, wx2, bx2,
            ws, bs, px)


def _node_weights(sp, E):
    v0, v1, v2 = sp["phi_v"]
    h0, h1, h2 = sp["phi_h"]
    z = jnp.zeros((16, 16), F32)
    wv1 = _bd(z.at[6:16].set(v0["W"]))
    bv1 = _tile(v0["b"])
    wv2, bv2 = _bd(v1["W"]), _tile(v1["b"])
    wvs = _bd(z.at[:, 3:6].set(jnp.broadcast_to(v2["W"], (16, 3))))
    bvs = _tile(jnp.zeros((16,), F32).at[3:6].set(v2["b"][0]))
    wh1n = _bd(z.at[6:16].set(h0["W"][0:10]))
    wh1m = _bd(h0["W"][10:26])
    bh1 = _tile(h0["b"])
    wh2, bh2 = _bd(h1["W"]), _tile(h1["b"])
    wh3 = _bd(jnp.zeros((16, 16), F32).at[:, 6:16].set(h2["W"]))
    bh3 = _tile(jnp.zeros((16,), F32).at[6:16].set(h2["b"]))
    pxt = jnp.asarray(_PXT) * (1.0 / (E - 1))
    shvx = jnp.asarray(_SHVX)
    gxv = jnp.asarray(_GXV)
    gh = jnp.asarray(_GH)
    s16 = (jnp.zeros((16,), F32).at[0:3].set(sp["cn_x"][0])
           .at[3:6].set(sp["cn_v"][0]).at[6:16].set(sp["ln_s"]))
    b16 = jnp.zeros((16,), F32).at[6:16].set(sp["ln_b"])
    sct, bft = _tile(s16), _tile(b16)
    return (wv1, bv1, wv2, bv2, wvs, bvs, wh1n, wh1m, bh1, wh2, bh2,
            wh3, bh3, pxt, shvx, gxv, gh, sct, bft)


def _edge_body(E, BK, sa_ref, ra_ref, wa, wb, wdt, gxall, c0, w2, b2, w3, b3,
               wx1, bx1, wx2, bx2, ws, bs, px, m_ref, ex_ref):
    pid = pl.program_id(0)
    sa = sa_ref[...]
    ra = ra_ref[...]
    diff = sa - ra
    d2 = _b16(_pdot(diff * diff, gxall[...]))
    l1 = (_dot(sa, wa[...]) + _dot(ra, wb[...])
          + d2 * wdt[...] + c0[...])
    m1 = jax.nn.gelu(l1)
    m2 = jax.nn.gelu(_dot(m1, w2[...]) + b2[...])
    m3 = _dot(m2, w3[...]) + b3[...]
    p1 = jax.nn.gelu(_dot(m3, wx1[...]) + bx1[...])
    p2 = jax.nn.gelu(_dot(p1, wx2[...]) + bx2[...])
    sb = _dot(p2, ws[...]) + bs[...]
    row = lax.broadcasted_iota(jnp.int32, (BK, 128), 0)
    lane = lax.broadcasted_iota(jnp.int32, (BK, 128), 1)
    eidx = (pid * BK + row) * P + lane // LN
    msk = (eidx < E).astype(F32)
    lane4 = lax.broadcasted_iota(jnp.int32, (BK, 32), 1)
    eidx4 = (pid * BK + lax.broadcasted_iota(jnp.int32, (BK, 32), 0)) * P + lane4 // 4
    msk4 = (eidx4 < E).astype(F32)
    m_ref[...] = m3 * msk
    ex_ref[...] = _pdot(diff * sb, px[...]) * msk4


def _edge_call(sa_p, ra_p, ew, E):
    E8 = sa_p.shape[0]
    BK = 4096 if E8 % 4096 == 0 else E8
    grid = E8 // BK
    row_spec = pl.BlockSpec((BK, 128), lambda i: (i, 0))

    def wspec(w):
        nd = w.ndim
        return pl.BlockSpec(w.shape, lambda i, _n=nd: (0,) * _n)

    return pl.pallas_call(
        functools.partial(_edge_body, E, BK),
        grid=(grid,),
        in_specs=[row_spec, row_spec] + [wspec(w) for w in ew],
        out_specs=[row_spec, pl.BlockSpec((BK, 32), lambda i: (i, 0))],
        out_shape=[jax.ShapeDtypeStruct((E8, 128), F32),
                   jax.ShapeDtypeStruct((E8, 32), F32)],
    )(sa_p, ra_p, *ew)


def _node_body(n_ref, m0_ref, m1_ref, e0_ref, e1_ref,
               wv1, bv1, wv2, bv2, wvs, bvs, wh1n, wh1m, bh1, wh2, bh2,
               wh3, bh3, pxt, shvx, gxv, gh, sct, bft, out_ref):
    nd = n_ref[...]
    lane = lax.broadcasted_iota(jnp.int32, nd.shape, 1) % 16
    mx = (lane < 3).astype(F32)
    mh = (lane >= 6).astype(F32)
    mi = m0_ref[...] + m1_ref[...]
    sx = _pdot(e0_ref[...] + e1_ref[...], pxt[...])
    v1 = jax.nn.gelu(_dot(nd, wv1[...]) + bv1[...])
    v2 = jax.nn.gelu(_dot(v1, wv2[...]) + bv2[...])
    vsc = _dot(v2, wvs[...]) + bvs[...]
    v_p = sx + vsc * nd
    x_p = nd * mx + _pdot(v_p, shvx[...])
    l1h = jax.nn.gelu(_dot(nd, wh1n[...]) + _dot(mi, wh1m[...]) + bh1[...])
    h2 = jax.nn.gelu(_dot(l1h, wh2[...]) + bh2[...])
    h_p = _dot(h2, wh3[...]) + bh3[...] + nd * mh
    u = x_p + v_p
    s2 = _pdot(u * u, gxv[...])
    den = jnp.maximum(jnp.sqrt(s2), 1e-8)
    uvn = u / den
    mu = _pdot(h_p, gh[...])
    dh = (h_p - mu) * mh
    var = _pdot(dh * dh, gh[...])
    rh = lax.rsqrt(var + 1e-6) * dh
    out_ref[...] = (uvn + rh) * sct[...] + bft[...]


def _node_call(n_p, m0, m1, e0, e1, nw_):
    N8 = n_p.shape[0]
    BKT = 1568
    N8p = -(-N8 // BKT) * BKT
    if N8p != N8:
        padrow = ((0, N8p - N8), (0, 0))
        n_p, m0, m1, e0, e1 = (jnp.pad(a, padrow)
                               for a in (n_p, m0, m1, e0, e1))
    BK = BKT
    grid = N8p // BKT
    row_spec = pl.BlockSpec((BK, 128), lambda i: (i, 0))
    ex_spec = pl.BlockSpec((BK, 32), lambda i: (i, 0))

    def wspec(w):
        nd = w.ndim
        return pl.BlockSpec(w.shape, lambda i, _n=nd: (0,) * _n)

    return pl.pallas_call(
        _node_body,
        grid=(grid,),
        in_specs=[row_spec, row_spec, row_spec, ex_spec, ex_spec]
                 + [wspec(w) for w in nw_],
        out_specs=[row_spec],
        out_shape=[jax.ShapeDtypeStruct((N8p, 128), F32)],
    )(n_p, m0, m1, e0, e1, *nw_)[0][:N8]


def _make_gather(N, C):
    KST = 8
    iters = C // (NW * KST)
    span = iters * KST
    mesh = plsc.VectorSubcoreMesh(core_axis_name="c", subcore_axis_name="s")

    @functools.partial(
        pl.kernel,
        out_type=(jax.ShapeDtypeStruct((C, 128, LN), F32),
                  jax.ShapeDtypeStruct((C, 128, LN), F32)),
        mesh=mesh,
        compiler_params=pltpu.CompilerParams(use_tc_tiling_on_sc=False),
        scratch_types=[
            pltpu.VMEM((KST, 128), jnp.int32),
            pltpu.VMEM((KST, 128), jnp.int32),
            pltpu.VMEM((KST, 128, LN), F32),
            pltpu.VMEM((KST, 128, LN), F32),
            pltpu.SemaphoreType.DMA,
            pltpu.SemaphoreType.DMA,
            pltpu.SemaphoreType.DMA,
        ],
    )
    def gather_k(nodes_hbm, s_hbm, r_hbm, out_s, out_r,
                 sidx, ridx, srow, rrow, sem_i, sem_g, sem_o):
        cid = lax.axis_index("c")
        tid = lax.axis_index("s")
        w = tid * 2 + cid
        start = w * span

        def body(i, carry):
            c0 = start + i * KST
            d1 = pltpu.async_copy(s_hbm.at[pl.ds(c0, KST)], sidx, sem_i)
            d2 = pltpu.async_copy(r_hbm.at[pl.ds(c0, KST)], ridx, sem_i)
            d1.wait()
            d2.wait()
            g = []
            for j in range(KST):
                g.append(pltpu.async_copy(nodes_hbm.at[sidx.at[j]],
                                          srow.at[j], sem_g))
                g.append(pltpu.async_copy(nodes_hbm.at[ridx.at[j]],
                                          rrow.at[j], sem_g))
            for d in g:
                d.wait()
            o1 = pltpu.async_copy(srow, out_s.at[pl.ds(c0, KST)], sem_o)
            o2 = pltpu.async_copy(rrow, out_r.at[pl.ds(c0, KST)], sem_o)
            o1.wait()
            o2.wait()
            return carry

        lax.fori_loop(0, iters, body, 0)

    return gather_k


def _make_scatter(N, C, D):
    KST = 8
    iters = C // (NW * KST)
    span = iters * KST
    rows_t = N // NS
    mesh = plsc.VectorSubcoreMesh(core_axis_name="c", subcore_axis_name="s")

    @functools.partial(
        pl.kernel,
        out_type=jax.ShapeDtypeStruct((2, N, D), F32),
        mesh=mesh,
        compiler_params=pltpu.CompilerParams(use_tc_tiling_on_sc=False),
        scratch_types=[
            pltpu.VMEM((KST, 128), jnp.int32),
            pltpu.VMEM((KST, 128, D), F32),
            pltpu.VMEM_SHARED((N, D), F32),
            pltpu.SemaphoreType.DMA,
            pltpu.SemaphoreType.DMA,
        ],
    )
    def scatter_k(v_hbm, r_hbm, z_hbm, v_out,
                  ridx, vrow, v_sh, sem_i, sem_s):
        cid = lax.axis_index("c")
        tid = lax.axis_index("s")
        w = tid * 2 + cid
        r0 = tid * rows_t
        pltpu.sync_copy(z_hbm, v_sh.at[pl.ds(r0, rows_t)])
        plsc.subcore_barrier()

        start = w * span

        def body(i, carry):
            c0 = start + i * KST
            d1 = pltpu.async_copy(r_hbm.at[pl.ds(c0, KST)], ridx, sem_i)
            d2 = pltpu.async_copy(v_hbm.at[pl.ds(c0, KST)], vrow, sem_i)
            d1.wait()
            d2.wait()
            sc = []
            for j in range(KST):
                sc.append(pltpu.async_copy(vrow.at[j], v_sh.at[ridx.at[j]],
                                           sem_s, add=True))
            for d in sc:
                d.wait()
            return carry

        lax.fori_loop(0, iters, body, 0)
        plsc.subcore_barrier()
        pltpu.sync_copy(v_sh.at[pl.ds(r0, rows_t)],
                        v_out.at[cid, pl.ds(r0, rows_t)])

    return scatter_k


def kernel(nodes, globals_, senders, receivers, params):
    N, F = nodes.shape
    E = senders.shape[0]
    UNIT = NW * 8 * 128
    E_pad = -(-E // UNIT) * UNIT
    C = E_pad // 128
    E8 = E_pad // P
    N8 = N // P
    gvec = globals_[0]
    s2d = jnp.pad(senders, (0, E_pad - E)).reshape(C, 128)
    r2d = jnp.pad(receivers, (0, E_pad - E)).reshape(C, 128)
    z16 = jnp.zeros((N // NS, LN), F32)
    z4 = jnp.zeros((N // NS, 4), F32)
    gather_k = _make_gather(N, C)
    scatter_m = _make_scatter(N, C, LN)
    scatter_x = _make_scatter(N, C, 4)
    for sp in params["steps"]:
        ew = _edge_weights(sp, gvec)
        nw_ = _node_weights(sp, E)
        out_s, out_r = gather_k(nodes, s2d, r2d)
        m_p, ex_p = _edge_call(out_s.reshape(E8, 128), out_r.reshape(E8, 128),
                               ew, E)
        m_sum = scatter_m(m_p.reshape(C, 128, LN), r2d, z16)
        ex_sum = scatter_x(ex_p.reshape(C, 128, 4), r2d, z4)
        nodes = _node_call(nodes.reshape(N8, 128),
                           m_sum[0].reshape(N8, 128), m_sum[1].reshape(N8, 128),
                           ex_sum[0].reshape(N8, 32), ex_sum[1].reshape(N8, 32),
                           nw_).reshape(N, LN)
    return nodes


# speedup vs baseline: 1.0066x; 1.0066x over previous
import functools

import numpy as np
import jax
import jax.numpy as jnp
from jax import lax
from jax.experimental import pallas as pl
from jax.experimental.pallas import tpu as pltpu
from jax.experimental.pallas import tpu_sc as plsc

F32 = jnp.float32
LN = 16
P = 8
NW = 32
NS = 16


def _dot(a, b):
    return lax.dot_general(a, b, (((1,), (0,)), ((), ())),
                           preferred_element_type=F32)


def _pdot(a, b):
    return lax.dot_general(a, b, (((1,), (0,)), ((), ())),
                           preferred_element_type=F32,
                           precision=lax.Precision.HIGHEST)


def _b16(x):
    return x.astype(jnp.bfloat16).astype(F32)


def _np_px():
    m = np.zeros((128, 32), np.float32)
    for j in range(P):
        for k in range(3):
            m[16 * j + k, 4 * j + k] = 1.0
    return m


def _np_pxt():
    m = np.zeros((32, 128), np.float32)
    for j in range(P):
        for k in range(3):
            m[4 * j + k, 16 * j + 3 + k] = 1.0
    return m


def _np_shvx():
    m = np.zeros((128, 128), np.float32)
    for j in range(P):
        for k in range(3):
            m[16 * j + 3 + k, 16 * j + k] = 1.0
    return m


def _np_gxv():
    m = np.zeros((128, 128), np.float32)
    for j in range(P):
        for a in range(3):
            for b in range(3):
                m[16 * j + a, 16 * j + b] = 1.0
                m[16 * j + 3 + a, 16 * j + 3 + b] = 1.0
    return m


def _np_gxall():
    m = np.zeros((128, 128), np.float32)
    for j in range(P):
        for a in range(3):
            for b in range(16):
                m[16 * j + a, 16 * j + b] = 1.0
    return m


def _np_gh():
    m = np.zeros((128, 128), np.float32)
    for j in range(P):
        for a in range(6, 16):
            for b in range(6, 16):
                m[16 * j + a, 16 * j + b] = 0.1
    return m


_PX = _np_px()
_GXALL = _np_gxall()
_PXT = _np_pxt()
_SHVX = _np_shvx()
_GXV = _np_gxv()
_GH = _np_gh()


def _bd(w16):
    return jnp.kron(jnp.eye(P, dtype=F32), w16.astype(F32))


def _tile(b16):
    return jnp.tile(b16.astype(F32), P)[None]


def _edge_weights(sp, gvec):
    e0, e1, e2 = sp["phi_e"]
    x0, x1, x2 = sp["phi_x"]
    W1 = e0["W"]
    z = jnp.zeros((16, 16), F32)
    wa = _bd(z.at[6:16].set(W1[0:10]))
    wb = _bd(z.at[6:16].set(W1[10:20]))
    wdt = _tile(_b16(W1[20]))
    gxall = jnp.asarray(_GXALL)
    c0 = _tile(e0["b"] + gvec @ W1[21:25])
    w2, b2 = _bd(e1["W"]), _tile(e1["b"])
    w3, b3 = _bd(e2["W"]), _tile(e2["b"])
    wx1, bx1 = _bd(x0["W"]), _tile(x0["b"])
    wx2, bx2 = _bd(x1["W"]), _tile(x1["b"])
    ws = _bd(z.at[:, 0:3].set(jnp.broadcast_to(x2["W"], (16, 3))))
    bs = _tile(jnp.zeros((16,), F32).at[0:3].set(x2["b"][0]))
    px = jnp.asarray(_PX)
    return (wa, wb, wdt, gxall, c0, w2, b2, w3, b3, wx1, bx1, wx2, bx2,
            ws, bs, px)


def _node_weights(sp, E):
    v0, v1, v2 = sp["phi_v"]
    h0, h1, h2 = sp["phi_h"]
    z = jnp.zeros((16, 16), F32)
    wv1 = _bd(z.at[6:16].set(v0["W"]))
    bv1 = _tile(v0["b"])
    wv2, bv2 = _bd(v1["W"]), _tile(v1["b"])
    wvs = _bd(z.at[:, 3:6].set(jnp.broadcast_to(v2["W"], (16, 3))))
    bvs = _tile(jnp.zeros((16,), F32).at[3:6].set(v2["b"][0]))
    wh1n = _bd(z.at[6:16].set(h0["W"][0:10]))
    wh1m = _bd(h0["W"][10:26])
    bh1 = _tile(h0["b"])
    wh2, bh2 = _bd(h1["W"]), _tile(h1["b"])
    wh3 = _bd(jnp.zeros((16, 16), F32).at[:, 6:16].set(h2["W"]))
    bh3 = _tile(jnp.zeros((16,), F32).at[6:16].set(h2["b"]))
    pxt = jnp.asarray(_PXT) * (1.0 / (E - 1))
    shvx = jnp.asarray(_SHVX)
    gxv = jnp.asarray(_GXV)
    gh = jnp.asarray(_GH)
    s16 = (jnp.zeros((16,), F32).at[0:3].set(sp["cn_x"][0])
           .at[3:6].set(sp["cn_v"][0]).at[6:16].set(sp["ln_s"]))
    b16 = jnp.zeros((16,), F32).at[6:16].set(sp["ln_b"])
    sct, bft = _tile(s16), _tile(b16)
    return (wv1, bv1, wv2, bv2, wvs, bvs, wh1n, wh1m, bh1, wh2, bh2,
            wh3, bh3, pxt, shvx, gxv, gh, sct, bft)


def _edge_body(E, BK, sa_ref, ra_ref, wa, wb, wdt, gxall, c0, w2, b2, w3, b3,
               wx1, bx1, wx2, bx2, ws, bs, px, m_ref, ex_ref):
    pid = pl.program_id(0)
    sa = sa_ref[...]
    ra = ra_ref[...]
    diff = sa - ra
    d2 = _b16(_pdot(diff * diff, gxall[...]))
    l1 = (_dot(sa, wa[...]) + _dot(ra, wb[...])
          + d2 * wdt[...] + c0[...])
    m1 = jax.nn.gelu(l1)
    m2 = jax.nn.gelu(_dot(m1, w2[...]) + b2[...])
    m3 = _dot(m2, w3[...]) + b3[...]
    p1 = jax.nn.gelu(_dot(m3, wx1[...]) + bx1[...])
    p2 = jax.nn.gelu(_dot(p1, wx2[...]) + bx2[...])
    sb = _dot(p2, ws[...]) + bs[...]
    row = lax.broadcasted_iota(jnp.int32, (BK, 128), 0)
    lane = lax.broadcasted_iota(jnp.int32, (BK, 128), 1)
    eidx = (pid * BK + row) * P + lane // LN
    msk = (eidx < E).astype(F32)
    lane4 = lax.broadcasted_iota(jnp.int32, (BK, 32), 1)
    eidx4 = (pid * BK + lax.broadcasted_iota(jnp.int32, (BK, 32), 0)) * P + lane4 // 4
    msk4 = (eidx4 < E).astype(F32)
    m_ref[...] = m3 * msk
    ex_ref[...] = _pdot(diff * sb, px[...]) * msk4


def _edge_call(sa_p, ra_p, ew, E):
    E8 = sa_p.shape[0]
    BK = 4096 if E8 % 4096 == 0 else E8
    grid = E8 // BK
    row_spec = pl.BlockSpec((BK, 128), lambda i: (i, 0))

    def wspec(w):
        nd = w.ndim
        return pl.BlockSpec(w.shape, lambda i, _n=nd: (0,) * _n)

    return pl.pallas_call(
        functools.partial(_edge_body, E, BK),
        grid=(grid,),
        in_specs=[row_spec, row_spec] + [wspec(w) for w in ew],
        out_specs=[row_spec, pl.BlockSpec((BK, 32), lambda i: (i, 0))],
        out_shape=[jax.ShapeDtypeStruct((E8, 128), F32),
                   jax.ShapeDtypeStruct((E8, 32), F32)],
    )(sa_p, ra_p, *ew)


def _node_body(n_ref, m0_ref, m1_ref, e0_ref, e1_ref,
               wv1, bv1, wv2, bv2, wvs, bvs, wh1n, wh1m, bh1, wh2, bh2,
               wh3, bh3, pxt, shvx, gxv, gh, sct, bft, out_ref):
    nd = n_ref[...]
    lane = lax.broadcasted_iota(jnp.int32, nd.shape, 1) % 16
    mx = (lane < 3).astype(F32)
    mh = (lane >= 6).astype(F32)
    mi = m0_ref[...] + m1_ref[...]
    sx = _pdot(e0_ref[...] + e1_ref[...], pxt[...])
    v1 = jax.nn.gelu(_dot(nd, wv1[...]) + bv1[...])
    v2 = jax.nn.gelu(_dot(v1, wv2[...]) + bv2[...])
    vsc = _dot(v2, wvs[...]) + bvs[...]
    v_p = sx + vsc * nd
    x_p = nd * mx + _pdot(v_p, shvx[...])
    l1h = jax.nn.gelu(_dot(nd, wh1n[...]) + _dot(mi, wh1m[...]) + bh1[...])
    h2 = jax.nn.gelu(_dot(l1h, wh2[...]) + bh2[...])
    h_p = _dot(h2, wh3[...]) + bh3[...] + nd * mh
    u = x_p + v_p
    s2 = _pdot(u * u, gxv[...])
    den = jnp.maximum(jnp.sqrt(s2), 1e-8)
    uvn = u / den
    mu = _pdot(h_p, gh[...])
    dh = (h_p - mu) * mh
    var = _pdot(dh * dh, gh[...])
    rh = lax.rsqrt(var + 1e-6) * dh
    out_ref[...] = (uvn + rh) * sct[...] + bft[...]


def _node_call(n_p, m0, m1, e0, e1, nw_):
    N8 = n_p.shape[0]
    BKT = 1568
    N8p = -(-N8 // BKT) * BKT
    if N8p != N8:
        padrow = ((0, N8p - N8), (0, 0))
        n_p, m0, m1, e0, e1 = (jnp.pad(a, padrow)
                               for a in (n_p, m0, m1, e0, e1))
    BK = BKT
    grid = N8p // BKT
    row_spec = pl.BlockSpec((BK, 128), lambda i: (i, 0))
    ex_spec = pl.BlockSpec((BK, 32), lambda i: (i, 0))

    def wspec(w):
        nd = w.ndim
        return pl.BlockSpec(w.shape, lambda i, _n=nd: (0,) * _n)

    return pl.pallas_call(
        _node_body,
        grid=(grid,),
        in_specs=[row_spec, row_spec, row_spec, ex_spec, ex_spec]
                 + [wspec(w) for w in nw_],
        out_specs=[row_spec],
        out_shape=[jax.ShapeDtypeStruct((N8p, 128), F32)],
    )(n_p, m0, m1, e0, e1, *nw_)[0][:N8]


def _make_gather(N, C):
    KST = 8
    iters = C // (NW * KST)
    span = iters * KST
    mesh = plsc.VectorSubcoreMesh(core_axis_name="c", subcore_axis_name="s")

    @functools.partial(
        pl.kernel,
        out_type=(jax.ShapeDtypeStruct((C, 128, LN), F32),
                  jax.ShapeDtypeStruct((C, 128, LN), F32)),
        mesh=mesh,
        compiler_params=pltpu.CompilerParams(use_tc_tiling_on_sc=False),
        scratch_types=[
            pltpu.VMEM((KST, 128), jnp.int32),
            pltpu.VMEM((KST, 128), jnp.int32),
            pltpu.VMEM((KST, 128, LN), F32),
            pltpu.VMEM((KST, 128, LN), F32),
            pltpu.SemaphoreType.DMA,
            pltpu.SemaphoreType.DMA,
            pltpu.SemaphoreType.DMA,
        ],
    )
    def gather_k(nodes_hbm, s_hbm, r_hbm, out_s, out_r,
                 sidx, ridx, srow, rrow, sem_i, sem_g, sem_o):
        cid = lax.axis_index("c")
        tid = lax.axis_index("s")
        w = tid * 2 + cid
        start = w * span

        def body(i, carry):
            c0 = start + i * KST
            d1 = pltpu.async_copy(s_hbm.at[pl.ds(c0, KST)], sidx, sem_i)
            d2 = pltpu.async_copy(r_hbm.at[pl.ds(c0, KST)], ridx, sem_i)
            d1.wait()
            d2.wait()
            g = []
            for j in range(KST):
                g.append(pltpu.async_copy(nodes_hbm.at[sidx.at[j]],
                                          srow.at[j], sem_g))
                g.append(pltpu.async_copy(nodes_hbm.at[ridx.at[j]],
                                          rrow.at[j], sem_g))
            for d in g:
                d.wait()
            o1 = pltpu.async_copy(srow, out_s.at[pl.ds(c0, KST)], sem_o)
            o2 = pltpu.async_copy(rrow, out_r.at[pl.ds(c0, KST)], sem_o)
            o1.wait()
            o2.wait()
            return carry

        lax.fori_loop(0, iters, body, 0)

    return gather_k


def _make_scatter(N, C, D):
    KST = 8
    iters = C // (NW * KST)
    span = iters * KST
    rows_t = N // NS
    mesh = plsc.VectorSubcoreMesh(core_axis_name="c", subcore_axis_name="s")

    @functools.partial(
        pl.kernel,
        out_type=jax.ShapeDtypeStruct((2, N, D), F32),
        mesh=mesh,
        compiler_params=pltpu.CompilerParams(use_tc_tiling_on_sc=False),
        scratch_types=[
            pltpu.VMEM((KST, 128), jnp.int32),
            pltpu.VMEM((KST, 128, D), F32),
            pltpu.VMEM_SHARED((N, D), F32),
            pltpu.SemaphoreType.DMA,
            pltpu.SemaphoreType.DMA,
        ],
    )
    def scatter_k(v_hbm, r_hbm, z_hbm, v_out,
                  ridx, vrow, v_sh, sem_i, sem_s):
        cid = lax.axis_index("c")
        tid = lax.axis_index("s")
        w = tid * 2 + cid
        r0 = tid * rows_t
        pltpu.sync_copy(z_hbm, v_sh.at[pl.ds(r0, rows_t)])
        plsc.subcore_barrier()

        start = w * span

        def body(i, carry):
            c0 = start + i * KST
            d1 = pltpu.async_copy(r_hbm.at[pl.ds(c0, KST)], ridx, sem_i)
            d2 = pltpu.async_copy(v_hbm.at[pl.ds(c0, KST)], vrow, sem_i)
            d1.wait()
            d2.wait()
            sc = []
            for j in range(KST):
                sc.append(pltpu.async_copy(vrow.at[j], v_sh.at[ridx.at[j]],
                                           sem_s, add=True))
            for d in sc:
                d.wait()
            return carry

        lax.fori_loop(0, iters, body, 0)
        plsc.subcore_barrier()
        pltpu.sync_copy(v_sh.at[pl.ds(r0, rows_t)],
                        v_out.at[cid, pl.ds(r0, rows_t)])

    return scatter_k


def kernel(nodes, globals_, senders, receivers, params):
    N, F = nodes.shape
    E = senders.shape[0]
    UNIT = NW * 8 * 128
    E_pad = -(-E // UNIT) * UNIT
    C = E_pad // 128
    E8 = E_pad // P
    NUNIT = NS * P * 98
    N_pad = -(-N // NUNIT) * NUNIT
    N8 = N_pad // P
    nodes = jnp.pad(nodes, ((0, N_pad - N), (0, 0)))
    gvec = globals_[0]
    s2d = jnp.pad(senders, (0, E_pad - E)).reshape(C, 128)
    r2d = jnp.pad(receivers, (0, E_pad - E)).reshape(C, 128)
    z16 = jnp.zeros((N_pad // NS, LN), F32)
    z4 = jnp.zeros((N_pad // NS, 4), F32)
    gather_k = _make_gather(N_pad, C)
    scatter_m = _make_scatter(N_pad, C, LN)
    scatter_x = _make_scatter(N_pad, C, 4)
    for sp in params["steps"]:
        ew = _edge_weights(sp, gvec)
        nw_ = _node_weights(sp, E)
        out_s, out_r = gather_k(nodes, s2d, r2d)
        m_p, ex_p = _edge_call(out_s.reshape(E8, 128), out_r.reshape(E8, 128),
                               ew, E)
        m_sum = scatter_m(m_p.reshape(C, 128, LN), r2d, z16)
        ex_sum = scatter_x(ex_p.reshape(C, 128, 4), r2d, z4)
        nodes = _node_call(nodes.reshape(N8, 128),
                           m_sum[0].reshape(N8, 128), m_sum[1].reshape(N8, 128),
                           ex_sum[0].reshape(N8, 32), ex_sum[1].reshape(N8, 32),
                           nw_).reshape(N_pad, LN)
    return nodes[:N]


# speedup vs baseline: 1.0066x; 1.0001x over previous
import functools

import numpy as np
import jax
import jax.numpy as jnp
from jax import lax
from jax.experimental import pallas as pl
from jax.experimental.pallas import tpu as pltpu
from jax.experimental.pallas import tpu_sc as plsc

F32 = jnp.float32
LN = 16
P = 8
NW = 32
NS = 16


def _dot(a, b):
    return lax.dot_general(a, b, (((1,), (0,)), ((), ())),
                           preferred_element_type=F32)


def _pdot(a, b):
    return lax.dot_general(a, b, (((1,), (0,)), ((), ())),
                           preferred_element_type=F32,
                           precision=lax.Precision.HIGHEST)


def _b16(x):
    return x.astype(jnp.bfloat16).astype(F32)


def _np_px():
    m = np.zeros((128, 32), np.float32)
    for j in range(P):
        for k in range(3):
            m[16 * j + k, 4 * j + k] = 1.0
    return m


def _np_pxt():
    m = np.zeros((32, 128), np.float32)
    for j in range(P):
        for k in range(3):
            m[4 * j + k, 16 * j + 3 + k] = 1.0
    return m


def _np_shvx():
    m = np.zeros((128, 128), np.float32)
    for j in range(P):
        for k in range(3):
            m[16 * j + 3 + k, 16 * j + k] = 1.0
    return m


def _np_gxv():
    m = np.zeros((128, 128), np.float32)
    for j in range(P):
        for a in range(3):
            for b in range(3):
                m[16 * j + a, 16 * j + b] = 1.0
                m[16 * j + 3 + a, 16 * j + 3 + b] = 1.0
    return m


def _np_gxall():
    m = np.zeros((128, 128), np.float32)
    for j in range(P):
        for a in range(3):
            for b in range(16):
                m[16 * j + a, 16 * j + b] = 1.0
    return m


def _np_gh():
    m = np.zeros((128, 128), np.float32)
    for j in range(P):
        for a in range(6, 16):
            for b in range(6, 16):
                m[16 * j + a, 16 * j + b] = 0.1
    return m


_PX = _np_px()
_GXALL = _np_gxall()
_PXT = _np_pxt()
_SHVX = _np_shvx()
_GXV = _np_gxv()
_GH = _np_gh()


def _bd(w16):
    return jnp.kron(jnp.eye(P, dtype=F32), w16.astype(F32))


def _tile(b16):
    return jnp.tile(b16.astype(F32), P)[None]


def _edge_weights(sp, gvec):
    e0, e1, e2 = sp["phi_e"]
    x0, x1, x2 = sp["phi_x"]
    W1 = e0["W"]
    z = jnp.zeros((16, 16), F32)
    wa = _bd(z.at[6:16].set(W1[0:10]))
    wb = _bd(z.at[6:16].set(W1[10:20]))
    wdt = _tile(_b16(W1[20]))
    gxall = jnp.asarray(_GXALL)
    c0 = _tile(e0["b"] + gvec @ W1[21:25])
    w2, b2 = _bd(e1["W"]), _tile(e1["b"])
    w3, b3 = _bd(e2["W"]), _tile(e2["b"])
    wx1, bx1 = _bd(x0["W"]), _tile(x0["b"])
    wx2, bx2 = _bd(x1["W"]), _tile(x1["b"])
    ws = _bd(z.at[:, 0:3].set(jnp.broadcast_to(x2["W"], (16, 3))))
    bs = _tile(jnp.zeros((16,), F32).at[0:3].set(x2["b"][0]))
    px = jnp.asarray(_PX)
    return (wa, wb, wdt, gxall, c0, w2, b2, w3, b3, wx1, bx1, wx2, bx2,
            ws, bs, px)


def _node_weights(sp, E):
    v0, v1, v2 = sp["phi_v"]
    h0, h1, h2 = sp["phi_h"]
    z = jnp.zeros((16, 16), F32)
    wv1 = _bd(z.at[6:16].set(v0["W"]))
    bv1 = _tile(v0["b"])
    wv2, bv2 = _bd(v1["W"]), _tile(v1["b"])
    wvs = _bd(z.at[:, 3:6].set(jnp.broadcast_to(v2["W"], (16, 3))))
    bvs = _tile(jnp.zeros((16,), F32).at[3:6].set(v2["b"][0]))
    wh1n = _bd(z.at[6:16].set(h0["W"][0:10]))
    wh1m = _bd(h0["W"][10:26])
    bh1 = _tile(h0["b"])
    wh2, bh2 = _bd(h1["W"]), _tile(h1["b"])
    wh3 = _bd(jnp.zeros((16, 16), F32).at[:, 6:16].set(h2["W"]))
    bh3 = _tile(jnp.zeros((16,), F32).at[6:16].set(h2["b"]))
    pxt = jnp.asarray(_PXT) * (1.0 / (E - 1))
    shvx = jnp.asarray(_SHVX)
    gxv = jnp.asarray(_GXV)
    gh = jnp.asarray(_GH)
    s16 = (jnp.zeros((16,), F32).at[0:3].set(sp["cn_x"][0])
           .at[3:6].set(sp["cn_v"][0]).at[6:16].set(sp["ln_s"]))
    b16 = jnp.zeros((16,), F32).at[6:16].set(sp["ln_b"])
    sct, bft = _tile(s16), _tile(b16)
    return (wv1, bv1, wv2, bv2, wvs, bvs, wh1n, wh1m, bh1, wh2, bh2,
            wh3, bh3, pxt, shvx, gxv, gh, sct, bft)


def _edge_body(E, BK, sa_ref, ra_ref, wa, wb, wdt, gxall, c0, w2, b2, w3, b3,
               wx1, bx1, wx2, bx2, ws, bs, px, m_ref, ex_ref):
    pid = pl.program_id(0)
    sa = sa_ref[...]
    ra = ra_ref[...]
    diff = sa - ra
    d2 = _b16(_pdot(diff * diff, gxall[...]))
    l1 = (_dot(sa, wa[...]) + _dot(ra, wb[...])
          + d2 * wdt[...] + c0[...])
    m1 = jax.nn.gelu(l1)
    m2 = jax.nn.gelu(_dot(m1, w2[...]) + b2[...])
    m3 = _dot(m2, w3[...]) + b3[...]
    p1 = jax.nn.gelu(_dot(m3, wx1[...]) + bx1[...])
    p2 = jax.nn.gelu(_dot(p1, wx2[...]) + bx2[...])
    sb = _dot(p2, ws[...]) + bs[...]
    row = lax.broadcasted_iota(jnp.int32, (BK, 128), 0)
    lane = lax.broadcasted_iota(jnp.int32, (BK, 128), 1)
    eidx = (pid * BK + row) * P + lane // LN
    msk = (eidx < E).astype(F32)
    lane4 = lax.broadcasted_iota(jnp.int32, (BK, 32), 1)
    eidx4 = (pid * BK + lax.broadcasted_iota(jnp.int32, (BK, 32), 0)) * P + lane4 // 4
    msk4 = (eidx4 < E).astype(F32)
    m_ref[...] = m3 * msk
    ex_ref[...] = _pdot(diff * sb, px[...]) * msk4


def _edge_call(sa_p, ra_p, ew, E):
    E8 = sa_p.shape[0]
    BK = 4096 if E8 % 4096 == 0 else E8
    grid = E8 // BK
    row_spec = pl.BlockSpec((BK, 128), lambda i: (i, 0))

    def wspec(w):
        nd = w.ndim
        return pl.BlockSpec(w.shape, lambda i, _n=nd: (0,) * _n)

    return pl.pallas_call(
        functools.partial(_edge_body, E, BK),
        grid=(grid,),
        in_specs=[row_spec, row_spec] + [wspec(w) for w in ew],
        out_specs=[row_spec, pl.BlockSpec((BK, 32), lambda i: (i, 0))],
        out_shape=[jax.ShapeDtypeStruct((E8, 128), F32),
                   jax.ShapeDtypeStruct((E8, 32), F32)],
    )(sa_p, ra_p, *ew)


def _node_body(n_ref, m0_ref, m1_ref, e0_ref, e1_ref,
               wv1, bv1, wv2, bv2, wvs, bvs, wh1n, wh1m, bh1, wh2, bh2,
               wh3, bh3, pxt, shvx, gxv, gh, sct, bft, out_ref):
    nd = n_ref[...]
    lane = lax.broadcasted_iota(jnp.int32, nd.shape, 1) % 16
    mx = (lane < 3).astype(F32)
    mh = (lane >= 6).astype(F32)
    mi = m0_ref[...] + m1_ref[...]
    sx = _pdot(e0_ref[...] + e1_ref[...], pxt[...])
    v1 = jax.nn.gelu(_dot(nd, wv1[...]) + bv1[...])
    v2 = jax.nn.gelu(_dot(v1, wv2[...]) + bv2[...])
    vsc = _dot(v2, wvs[...]) + bvs[...]
    v_p = sx + vsc * nd
    x_p = nd * mx + _pdot(v_p, shvx[...])
    l1h = jax.nn.gelu(_dot(nd, wh1n[...]) + _dot(mi, wh1m[...]) + bh1[...])
    h2 = jax.nn.gelu(_dot(l1h, wh2[...]) + bh2[...])
    h_p = _dot(h2, wh3[...]) + bh3[...] + nd * mh
    u = x_p + v_p
    s2 = _pdot(u * u, gxv[...])
    den = jnp.maximum(jnp.sqrt(s2), 1e-8)
    uvn = u / den
    mu = _pdot(h_p, gh[...])
    dh = (h_p - mu) * mh
    var = _pdot(dh * dh, gh[...])
    rh = lax.rsqrt(var + 1e-6) * dh
    out_ref[...] = (uvn + rh) * sct[...] + bft[...]


def _node_call(n_p, m0, m1, e0, e1, nw_):
    N8 = n_p.shape[0]
    BKT = 1568
    N8p = -(-N8 // BKT) * BKT
    if N8p != N8:
        padrow = ((0, N8p - N8), (0, 0))
        n_p, m0, m1, e0, e1 = (jnp.pad(a, padrow)
                               for a in (n_p, m0, m1, e0, e1))
    BK = BKT
    grid = N8p // BKT
    row_spec = pl.BlockSpec((BK, 128), lambda i: (i, 0))
    ex_spec = pl.BlockSpec((BK, 32), lambda i: (i, 0))

    def wspec(w):
        nd = w.ndim
        return pl.BlockSpec(w.shape, lambda i, _n=nd: (0,) * _n)

    return pl.pallas_call(
        _node_body,
        grid=(grid,),
        in_specs=[row_spec, row_spec, row_spec, ex_spec, ex_spec]
                 + [wspec(w) for w in nw_],
        out_specs=[row_spec],
        out_shape=[jax.ShapeDtypeStruct((N8p, 128), F32)],
    )(n_p, m0, m1, e0, e1, *nw_)[0][:N8]


def _make_gather(N, C):
    KST = 8
    iters = C // (NW * KST)
    span = iters * KST
    mesh = plsc.VectorSubcoreMesh(core_axis_name="c", subcore_axis_name="s")

    @functools.partial(
        pl.kernel,
        out_type=(jax.ShapeDtypeStruct((C * 128, LN), F32),
                  jax.ShapeDtypeStruct((C * 128, LN), F32)),
        mesh=mesh,
        compiler_params=pltpu.CompilerParams(use_tc_tiling_on_sc=False),
        scratch_types=[
            pltpu.VMEM((KST, 128), jnp.int32),
            pltpu.VMEM((KST, 128), jnp.int32),
            pltpu.VMEM((KST * 128, LN), F32),
            pltpu.VMEM((KST * 128, LN), F32),
            pltpu.SemaphoreType.DMA,
            pltpu.SemaphoreType.DMA,
            pltpu.SemaphoreType.DMA,
        ],
    )
    def gather_k(nodes_hbm, s_hbm, r_hbm, out_s, out_r,
                 sidx, ridx, srow, rrow, sem_i, sem_g, sem_o):
        cid = lax.axis_index("c")
        tid = lax.axis_index("s")
        w = tid * 2 + cid
        start = w * span

        def body(i, carry):
            c0 = start + i * KST
            d1 = pltpu.async_copy(s_hbm.at[pl.ds(c0, KST)], sidx, sem_i)
            d2 = pltpu.async_copy(r_hbm.at[pl.ds(c0, KST)], ridx, sem_i)
            d1.wait()
            d2.wait()
            g = []
            for j in range(KST):
                g.append(pltpu.async_copy(nodes_hbm.at[sidx.at[j]],
                                          srow.at[pl.ds(j * 128, 128)],
                                          sem_g))
                g.append(pltpu.async_copy(nodes_hbm.at[ridx.at[j]],
                                          rrow.at[pl.ds(j * 128, 128)],
                                          sem_g))
            for d in g:
                d.wait()
            o1 = pltpu.async_copy(srow, out_s.at[pl.ds(c0 * 128, KST * 128)],
                                  sem_o)
            o2 = pltpu.async_copy(rrow, out_r.at[pl.ds(c0 * 128, KST * 128)],
                                  sem_o)
            o1.wait()
            o2.wait()
            return carry

        lax.fori_loop(0, iters, body, 0)

    return gather_k


def _make_scatter(N, C, D):
    KST = 8
    iters = C // (NW * KST)
    span = iters * KST
    rows_t = N // NS
    mesh = plsc.VectorSubcoreMesh(core_axis_name="c", subcore_axis_name="s")

    @functools.partial(
        pl.kernel,
        out_type=jax.ShapeDtypeStruct((2, N, D), F32),
        mesh=mesh,
        compiler_params=pltpu.CompilerParams(use_tc_tiling_on_sc=False),
        scratch_types=[
            pltpu.VMEM((KST, 128), jnp.int32),
            pltpu.VMEM((KST, 128, D), F32),
            pltpu.VMEM_SHARED((N, D), F32),
            pltpu.SemaphoreType.DMA,
            pltpu.SemaphoreType.DMA,
        ],
    )
    def scatter_k(v_hbm, r_hbm, z_hbm, v_out,
                  ridx, vrow, v_sh, sem_i, sem_s):
        cid = lax.axis_index("c")
        tid = lax.axis_index("s")
        w = tid * 2 + cid
        r0 = tid * rows_t
        pltpu.sync_copy(z_hbm, v_sh.at[pl.ds(r0, rows_t)])
        plsc.subcore_barrier()

        start = w * span

        def body(i, carry):
            c0 = start + i * KST
            d1 = pltpu.async_copy(r_hbm.at[pl.ds(c0, KST)], ridx, sem_i)
            d2 = pltpu.async_copy(v_hbm.at[pl.ds(c0, KST)], vrow, sem_i)
            d1.wait()
            d2.wait()
            sc = []
            for j in range(KST):
                sc.append(pltpu.async_copy(vrow.at[j], v_sh.at[ridx.at[j]],
                                           sem_s, add=True))
            for d in sc:
                d.wait()
            return carry

        lax.fori_loop(0, iters, body, 0)
        plsc.subcore_barrier()
        pltpu.sync_copy(v_sh.at[pl.ds(r0, rows_t)],
                        v_out.at[cid, pl.ds(r0, rows_t)])

    return scatter_k


def kernel(nodes, globals_, senders, receivers, params):
    N, F = nodes.shape
    E = senders.shape[0]
    UNIT = NW * 8 * 128
    E_pad = -(-E // UNIT) * UNIT
    C = E_pad // 128
    E8 = E_pad // P
    NUNIT = NS * P * 98
    N_pad = -(-N // NUNIT) * NUNIT
    N8 = N_pad // P
    nodes = jnp.pad(nodes, ((0, N_pad - N), (0, 0)))
    gvec = globals_[0]
    s2d = jnp.pad(senders, (0, E_pad - E)).reshape(C, 128)
    r2d = jnp.pad(receivers, (0, E_pad - E)).reshape(C, 128)
    z16 = jnp.zeros((N_pad // NS, LN), F32)
    z4 = jnp.zeros((N_pad // NS, 4), F32)
    gather_k = _make_gather(N_pad, C)
    scatter_m = _make_scatter(N_pad, C, LN)
    scatter_x = _make_scatter(N_pad, C, 4)
    for sp in params["steps"]:
        ew = _edge_weights(sp, gvec)
        nw_ = _node_weights(sp, E)
        out_s, out_r = gather_k(nodes, s2d, r2d)
        m_p, ex_p = _edge_call(out_s.reshape(E8, 128), out_r.reshape(E8, 128),
                               ew, E)
        m_sum = scatter_m(m_p.reshape(C, 128, LN), r2d, z16)
        ex_sum = scatter_x(ex_p.reshape(C, 128, 4), r2d, z4)
        nodes = _node_call(nodes.reshape(N8, 128),
                           m_sum[0].reshape(N8, 128), m_sum[1].reshape(N8, 128),
                           ex_sum[0].reshape(N8, 32), ex_sum[1].reshape(N8, 32),
                           nw_).reshape(N_pad, LN)
    return nodes[:N]
